# baseline (speedup 1.0000x reference)
import functools

import jax
import jax.numpy as jnp
from jax import lax
from jax.experimental import pallas as pl
from jax.experimental.pallas import tpu as pltpu
from jax.experimental.pallas import tpu_sc as plsc

N = 10000
E = 320000
D = 128
EPS_BN = 1e-5

NC = 2
NS = 16
NW = NC * NS
EPW = E // NW
K = 80
T = EPW // K
BPT = 624
REM = N - NS * BPT


def _sc_body(z_hbm, src_hbm, dst_hbm, ew_hbm, out_hbm,
             acc, srcv, dstv, roww, eww,
             sg0, sg1, se0, se1,
             si0, si1, si2, si3, sd0, sd1, sd2, sd3, ss0, ss1):
    c = lax.axis_index("c")
    s = lax.axis_index("s")
    SG, SE, SS = (sg0, sg1), (se0, se1), (ss0, ss1)
    SI, SD = (si0, si1, si2, si3), (sd0, sd1, sd2, sd3)
    base0 = (c * NS + s) * EPW
    base0h = (c * NS + s) * (EPW // 2)

    def ld_src(j, q):
        pltpu.async_copy(src_hbm.at[pl.ds(base0 + j * K, K)], srcv.at[q], SI[q])

    def wait_src(q):
        pltpu.make_async_copy(src_hbm.at[pl.ds(0, K)], srcv.at[q], SI[q]).wait()

    def ld_dst(j, q):
        pltpu.async_copy(dst_hbm.at[pl.ds(base0 + j * K, K)], dstv.at[q], SD[q])

    def wait_dst(q):
        pltpu.make_async_copy(dst_hbm.at[pl.ds(0, K)], dstv.at[q], SD[q]).wait()

    def ld_ew(j, b):
        pltpu.async_copy(
            ew_hbm.at[pl.ds(base0h + j * (K // 2), K // 2), :], eww.at[b], SE[b])

    def wait_ew(b):
        pltpu.make_async_copy(
            ew_hbm.at[pl.ds(0, K // 2), :], eww.at[b], SE[b]).wait()

    def gather(q, b):
        pltpu.async_copy(z_hbm.at[srcv.at[q]], roww.at[b], SG[b])

    def wait_gather(q, b):
        pltpu.make_async_copy(z_hbm.at[srcv.at[q]], roww.at[b], SG[b]).wait()

    def scat(q, b):
        pltpu.async_copy(roww.at[b], acc.at[dstv.at[q]], SS[b], add=True)

    def wait_scat(q, b):
        pltpu.make_async_copy(roww.at[b], acc.at[dstv.at[q]], SS[b]).wait()

    def compute(b):
        hi = jnp.int32(-65536)

        def _rows(r, rc):
            for rr in range(2):
                pr = r * 2 + rr
                e0 = pr * 2
                for i in range(D // 16):
                    sl = pl.ds(i * 16, 16)
                    w = eww[b, pr, sl]
                    ea = lax.bitcast_convert_type(w << 16, jnp.float32)
                    eb = lax.bitcast_convert_type(w & hi, jnp.float32)
                    roww[b, e0, sl] = jnp.maximum(roww[b, e0, sl] + ea, 0.0)
                    roww[b, e0 + 1, sl] = jnp.maximum(
                        roww[b, e0 + 1, sl] + eb, 0.0)
            return rc

        lax.fori_loop(0, K // 4, _rows, 0)

    def slot(j, q, dyn_guard, do_dst, do_src, do_gather, do_ew, do_scwait):
        b = q % 2
        nb = (b + 1) % 2

        def _guarded(pred, fn):
            if dyn_guard:
                pl.when(pred)(fn)
            elif pred:
                fn()

        if do_dst:
            _guarded(j < T - 2 if dyn_guard else True,
                     lambda: ld_dst(j + 2, (q + 2) % 4))
        if do_src:
            _guarded(j < T - 3 if dyn_guard else True,
                     lambda: ld_src(j + 3, (q + 3) % 4))
        wait_gather(q, b)
        if do_scwait:
            _guarded(j >= 1 if dyn_guard else True,
                     lambda: wait_scat((q + 3) % 4, nb))
        if do_gather:
            def _next_gather():
                wait_src((q + 1) % 4)
                gather((q + 1) % 4, nb)
            _guarded(j < T - 1 if dyn_guard else True, _next_gather)
        wait_ew(b)
        compute(b)
        if do_ew:
            _guarded(j < T - 2 if dyn_guard else True,
                     lambda: ld_ew(j + 2, b))
        wait_dst(q)
        scat(q, b)

    ld_src(0, 0)
    ld_src(1, 1)
    ld_src(2, 2)
    ld_dst(0, 0)
    ld_dst(1, 1)
    ld_ew(0, 0)
    ld_ew(1, 1)
    wait_src(0)
    gather(0, 0)

    zero16 = jnp.zeros((16,), jnp.float32)

    def _zrow(r, carry):
        for i in range(D // 16):
            roww[1, r, pl.ds(i * 16, 16)] = zero16
        return carry

    lax.fori_loop(0, K, _zrow, 0)
    for kk in range(BPT // K):
        pltpu.sync_copy(roww.at[1], acc.at[pl.ds(s * BPT + kk * K, K), :])
    rem_r = BPT - (BPT // K) * K
    pltpu.sync_copy(roww.at[1, pl.ds(0, rem_r), :],
                    acc.at[pl.ds(s * BPT + (BPT // K) * K, rem_r), :])

    @pl.when(s == NS - 1)
    def _():
        pltpu.sync_copy(roww.at[1, pl.ds(0, REM), :],
                        acc.at[pl.ds(NS * BPT, REM), :])

    plsc.subcore_barrier()

    def _quad(g, carry):
        for u in range(4):
            slot(g * 4 + u, u, True, True, True, True, True, True)
        return carry

    lax.fori_loop(0, T // 4, _quad, 0)
    for j in range((T // 4) * 4, T):
        slot(j, j % 4, False, j < T - 2, j < T - 3, j < T - 1, j < T - 2,
             j >= 1)

    wait_scat((T - 1) % 4, (T - 1) % 2)
    plsc.subcore_barrier()

    pltpu.sync_copy(acc.at[pl.ds(s * BPT, BPT), :],
                    out_hbm.at[pl.ds(c * N + s * BPT, BPT), :])

    @pl.when(s == NS - 1)
    def _():
        pltpu.sync_copy(acc.at[pl.ds(NS * BPT, REM), :],
                        out_hbm.at[pl.ds(c * N + NS * BPT, REM), :])


@functools.cache
def _sc_msgpass_fn():
    mesh = plsc.VectorSubcoreMesh(core_axis_name="c", subcore_axis_name="s",
                                  num_cores=NC, num_subcores=NS)
    return pl.kernel(
        _sc_body,
        out_type=jax.ShapeDtypeStruct((NC * N, D), jnp.float32),
        mesh=mesh,
        scratch_types=[
            pltpu.MemorySpace.VMEM_SHARED((N, D), jnp.float32),
            pltpu.MemorySpace.VMEM((4, K), jnp.int32),
            pltpu.MemorySpace.VMEM((4, K), jnp.int32),
            pltpu.MemorySpace.VMEM((2, K, D), jnp.float32),
            pltpu.MemorySpace.VMEM((2, K // 2, D), jnp.int32),
        ] + [pltpu.SemaphoreType.DMA] * 14,
    )


def _mlp_body(z_ref, p0_ref, p1_ref, w1_ref, b1_ref, w2_ref, b2_ref, o_ref):
    h = z_ref[...] + p0_ref[...] + p1_ref[...]
    h = jnp.dot(h, w1_ref[...], preferred_element_type=jnp.float32,
                precision=lax.Precision.HIGHEST) + b1_ref[...]
    h = jnp.maximum(h, 0.0)
    h = jnp.dot(h, w2_ref[...], preferred_element_type=jnp.float32,
                precision=lax.Precision.HIGHEST) + b2_ref[...]
    o_ref[...] = jnp.maximum(h, 0.0)


def _pack_body(x_ref, o_ref):
    u = lax.bitcast_convert_type(x_ref[...], jnp.uint32)
    a = u[:, :D]
    b = u[:, D:]
    ra = (a + jnp.uint32(0x7FFF) + ((a >> 16) & jnp.uint32(1))) >> 16
    rb = (b + jnp.uint32(0x7FFF) + ((b >> 16) & jnp.uint32(1))) >> 16
    o_ref[...] = lax.bitcast_convert_type(ra | (rb << 16), jnp.int32)


_PBLK = 2000


def _pack_ew(edge_weight):
    ew2 = edge_weight.reshape(E // 2, 2 * D)
    return pl.pallas_call(
        _pack_body,
        grid=(E // 2 // _PBLK,),
        in_specs=[pl.BlockSpec((_PBLK, 2 * D), lambda i: (i, 0))],
        out_specs=pl.BlockSpec((_PBLK, D), lambda i: (i, 0)),
        out_shape=jax.ShapeDtypeStruct((E // 2, D), jnp.int32),
    )(ew2)


_BLK = 1000


def _mlp_call(z, pp, w1, b1, w2, b2):
    row_spec = pl.BlockSpec((_BLK, D), lambda i: (i, 0))
    full = pl.BlockSpec((D, D), lambda i: (0, 0))
    vec = pl.BlockSpec((1, D), lambda i: (0, 0))
    return pl.pallas_call(
        _mlp_body,
        grid=(N // _BLK,),
        in_specs=[
            row_spec,
            pl.BlockSpec((_BLK, D), lambda i: (i, 0)),
            pl.BlockSpec((_BLK, D), lambda i: (i + N // _BLK, 0)),
            full, vec, full, vec,
        ],
        out_specs=row_spec,
        out_shape=jax.ShapeDtypeStruct((N, D), jnp.float32),
    )(z, pp, pp, w1, b1, w2, b2)


def kernel(x, edge_index, edge_weight,
           W1_0, b1_0, g_0, be_0, W2_0, b2_0,
           W1_1, b1_1, g_1, be_1, W2_1, b2_1,
           W1_2, b1_2, g_2, be_2, W2_2, b2_2):
    src = edge_index[0]
    dst = edge_index[1]
    ewb = _pack_ew(edge_weight)
    inv = 1.0 / jnp.sqrt(1.0 + EPS_BN)
    z = x
    for (W1, b1, g, be, W2, b2) in (
        (W1_0, b1_0, g_0, be_0, W2_0, b2_0),
        (W1_1, b1_1, g_1, be_1, W2_1, b2_1),
        (W1_2, b1_2, g_2, be_2, W2_2, b2_2),
    ):
        scale = g * inv
        w1f = W1 * scale[None, :]
        b1f = (b1 * scale + be).reshape(1, D)
        pp = _sc_msgpass_fn()(z, src, dst, ewb)
        z = _mlp_call(z, pp, w1f, b1f, W2, b2.reshape(1, D))
    return z


# speedup vs baseline: 2.3843x; 2.3843x over previous
import functools

import jax
import jax.numpy as jnp
from jax import lax
from jax.experimental import pallas as pl
from jax.experimental.pallas import tpu as pltpu
from jax.experimental.pallas import tpu_sc as plsc

N = 10000
E = 320000
D = 128
EPS_BN = 1e-5

NC = 2
NS = 16
NW = NC * NS
EPW = E // NW
K = 80
T = EPW // K
BPT = 624
REM = N - NS * BPT


def _sc_body(z_hbm, src_hbm, dst_hbm, ew_hbm, out_hbm,
             acc, srcv, dstv, roww, eww,
             sg0, sg1, se0, se1,
             si0, si1, si2, si3, sd0, sd1, sd2, sd3, ss0, ss1):
    c = lax.axis_index("c")
    s = lax.axis_index("s")
    SG, SE, SS = (sg0, sg1), (se0, se1), (ss0, ss1)
    SI, SD = (si0, si1, si2, si3), (sd0, sd1, sd2, sd3)
    base0 = (c * NS + s) * EPW

    def ld_src(j, q):
        pltpu.async_copy(src_hbm.at[pl.ds(base0 + j * K, K)], srcv.at[q], SI[q])

    def wait_src(q):
        pltpu.make_async_copy(src_hbm.at[pl.ds(0, K)], srcv.at[q], SI[q]).wait()

    def ld_dst(j, q):
        pltpu.async_copy(dst_hbm.at[pl.ds(base0 + j * K, K)], dstv.at[q], SD[q])

    def wait_dst(q):
        pltpu.make_async_copy(dst_hbm.at[pl.ds(0, K)], dstv.at[q], SD[q]).wait()

    def ld_ew(j, b):
        pltpu.async_copy(ew_hbm.at[pl.ds(base0 + j * K, K), :], eww.at[b], SE[b])

    def wait_ew(b):
        pltpu.make_async_copy(ew_hbm.at[pl.ds(0, K), :], eww.at[b], SE[b]).wait()

    def gather(q, b):
        pltpu.async_copy(z_hbm.at[srcv.at[q]], roww.at[b], SG[b])

    def wait_gather(q, b):
        pltpu.make_async_copy(z_hbm.at[srcv.at[q]], roww.at[b], SG[b]).wait()

    def scat(q, b):
        pltpu.async_copy(roww.at[b], acc.at[dstv.at[q]], SS[b], add=True)

    def wait_scat(q, b):
        pltpu.make_async_copy(roww.at[b], acc.at[dstv.at[q]], SS[b]).wait()

    def compute(b):
        def _rows(r, rc):
            for rr in range(4):
                for i in range(D // 16):
                    sl = pl.ds(i * 16, 16)
                    roww[b, r * 4 + rr, sl] = jnp.maximum(
                        roww[b, r * 4 + rr, sl] + eww[b, r * 4 + rr, sl], 0.0)
            return rc

        lax.fori_loop(0, K // 4, _rows, 0)

    def slot(j, q, dyn_guard, do_dst, do_src, do_gather, do_ew, do_scwait):
        b = q % 2
        nb = (b + 1) % 2

        def _guarded(pred, fn):
            if dyn_guard:
                pl.when(pred)(fn)
            elif pred:
                fn()

        if do_dst:
            _guarded(j < T - 2 if dyn_guard else True,
                     lambda: ld_dst(j + 2, (q + 2) % 4))
        if do_src:
            _guarded(j < T - 3 if dyn_guard else True,
                     lambda: ld_src(j + 3, (q + 3) % 4))
        wait_gather(q, b)
        if do_scwait:
            _guarded(j >= 1 if dyn_guard else True,
                     lambda: wait_scat((q + 3) % 4, nb))
        if do_gather:
            def _next_gather():
                wait_src((q + 1) % 4)
                gather((q + 1) % 4, nb)
            _guarded(j < T - 1 if dyn_guard else True, _next_gather)
        wait_ew(b)
        compute(b)
        if do_ew:
            _guarded(j < T - 2 if dyn_guard else True,
                     lambda: ld_ew(j + 2, b))
        wait_dst(q)
        scat(q, b)

    ld_src(0, 0)
    ld_src(1, 1)
    ld_src(2, 2)
    ld_dst(0, 0)
    ld_dst(1, 1)
    ld_ew(0, 0)
    ld_ew(1, 1)
    wait_src(0)
    gather(0, 0)

    zero16 = jnp.zeros((16,), jnp.float32)

    def _zrow(r, carry):
        for i in range(D // 16):
            roww[1, r, pl.ds(i * 16, 16)] = zero16
        return carry

    lax.fori_loop(0, K, _zrow, 0)
    for kk in range(BPT // K):
        pltpu.sync_copy(roww.at[1], acc.at[pl.ds(s * BPT + kk * K, K), :])
    rem_r = BPT - (BPT // K) * K
    pltpu.sync_copy(roww.at[1, pl.ds(0, rem_r), :],
                    acc.at[pl.ds(s * BPT + (BPT // K) * K, rem_r), :])

    @pl.when(s == NS - 1)
    def _():
        pltpu.sync_copy(roww.at[1, pl.ds(0, REM), :],
                        acc.at[pl.ds(NS * BPT, REM), :])

    plsc.subcore_barrier()

    def _quad(g, carry):
        for u in range(4):
            slot(g * 4 + u, u, True, True, True, True, True, True)
        return carry

    lax.fori_loop(0, T // 4, _quad, 0)
    for j in range((T // 4) * 4, T):
        slot(j, j % 4, False, j < T - 2, j < T - 3, j < T - 1, j < T - 2,
             j >= 1)

    wait_scat((T - 1) % 4, (T - 1) % 2)
    plsc.subcore_barrier()

    pltpu.sync_copy(acc.at[pl.ds(s * BPT, BPT), :],
                    out_hbm.at[pl.ds(c * N + s * BPT, BPT), :])

    @pl.when(s == NS - 1)
    def _():
        pltpu.sync_copy(acc.at[pl.ds(NS * BPT, REM), :],
                        out_hbm.at[pl.ds(c * N + NS * BPT, REM), :])


@functools.cache
def _sc_msgpass_fn():
    mesh = plsc.VectorSubcoreMesh(core_axis_name="c", subcore_axis_name="s",
                                  num_cores=NC, num_subcores=NS)
    return pl.kernel(
        _sc_body,
        out_type=jax.ShapeDtypeStruct((NC * N, D), jnp.float32),
        mesh=mesh,
        scratch_types=[
            pltpu.MemorySpace.VMEM_SHARED((N, D), jnp.float32),
            pltpu.MemorySpace.VMEM((4, K), jnp.int32),
            pltpu.MemorySpace.VMEM((4, K), jnp.int32),
            pltpu.MemorySpace.VMEM((2, K, D), jnp.float32),
            pltpu.MemorySpace.VMEM((2, K, D), jnp.float32),
        ] + [pltpu.SemaphoreType.DMA] * 14,
    )


def _mlp_body(z_ref, p0_ref, p1_ref, w1_ref, b1_ref, w2_ref, b2_ref, o_ref):
    h = z_ref[...] + p0_ref[...] + p1_ref[...]
    h = jnp.dot(h, w1_ref[...],
                preferred_element_type=jnp.float32) + b1_ref[...]
    h = jnp.maximum(h, 0.0)
    h = jnp.dot(h, w2_ref[...],
                preferred_element_type=jnp.float32) + b2_ref[...]
    o_ref[...] = jnp.maximum(h, 0.0)


_BLK = 1000


def _mlp_call(z, pp, w1, b1, w2, b2):
    row_spec = pl.BlockSpec((_BLK, D), lambda i: (i, 0))
    full = pl.BlockSpec((D, D), lambda i: (0, 0))
    vec = pl.BlockSpec((1, D), lambda i: (0, 0))
    return pl.pallas_call(
        _mlp_body,
        grid=(N // _BLK,),
        in_specs=[
            row_spec,
            pl.BlockSpec((_BLK, D), lambda i: (i, 0)),
            pl.BlockSpec((_BLK, D), lambda i: (i + N // _BLK, 0)),
            full, vec, full, vec,
        ],
        out_specs=row_spec,
        out_shape=jax.ShapeDtypeStruct((N, D), jnp.float32),
    )(z, pp, pp, w1, b1, w2, b2)


def kernel(x, edge_index, edge_weight,
           W1_0, b1_0, g_0, be_0, W2_0, b2_0,
           W1_1, b1_1, g_1, be_1, W2_1, b2_1,
           W1_2, b1_2, g_2, be_2, W2_2, b2_2):
    src = edge_index[0]
    dst = edge_index[1]
    inv = 1.0 / jnp.sqrt(1.0 + EPS_BN)
    z = x
    for (W1, b1, g, be, W2, b2) in (
        (W1_0, b1_0, g_0, be_0, W2_0, b2_0),
        (W1_1, b1_1, g_1, be_1, W2_1, b2_1),
        (W1_2, b1_2, g_2, be_2, W2_2, b2_2),
    ):
        scale = g * inv
        w1f = W1 * scale[None, :]
        b1f = (b1 * scale + be).reshape(1, D)
        pp = _sc_msgpass_fn()(z, src, dst, edge_weight)
        z = _mlp_call(z, pp, w1f, b1f, W2, b2.reshape(1, D))
    return z
